# Initial kernel scaffold; baseline (speedup 1.0000x reference)
import jax
import jax.numpy as jnp
from jax.experimental import pallas as pl


def kernel(x, batch, v):
    raise NotImplementedError("write your pallas kernel here")


# speedup vs baseline: 72.6994x; 72.6994x over previous
import functools

import jax
import jax.numpy as jnp
from jax.experimental import pallas as pl

_BUMP_STEPS = 32
_RADIUS = 1.1
_SCALE = 100.0
_NUM_SEGMENTS = 16
_NUM_THETAS = 32
_CHUNK = 2000


def _ect_kernel(x_ref, b_ref, v_ref, lin_ref, out_ref):
    i = pl.program_id(0)

    @pl.when(i == 0)
    def _init():
        out_ref[...] = jnp.zeros_like(out_ref)

    nh = jnp.dot(x_ref[...], v_ref[...], preferred_element_type=jnp.float32)
    nh_tiled = jnp.tile(nh, (1, _BUMP_STEPS))
    ecc = jax.nn.sigmoid(lin_ref[...] - nh_tiled)
    seg = b_ref[...]
    oh = (seg == jax.lax.broadcasted_iota(jnp.int32, (1, _NUM_SEGMENTS), 1)
          ).astype(jnp.float32)
    partial = jax.lax.dot_general(
        oh, ecc, (((0,), (0,)), ((), ())),
        preferred_element_type=jnp.float32)
    out_ref[...] += partial


@jax.jit
def kernel(x, batch, v):
    n = x.shape[0]
    grid = n // _CHUNK
    st = _BUMP_STEPS * _NUM_THETAS
    lin = jnp.linspace(-_RADIUS, _RADIUS, _BUMP_STEPS, dtype=jnp.float32)
    lin_row = (_SCALE * jnp.repeat(lin, _NUM_THETAS)).reshape(1, st)
    v_scaled = (v * _SCALE).astype(jnp.float32)
    batch2d = batch.reshape(n, 1)
    out = pl.pallas_call(
        _ect_kernel,
        grid=(grid,),
        in_specs=[
            pl.BlockSpec((_CHUNK, x.shape[1]), lambda i: (i, 0)),
            pl.BlockSpec((_CHUNK, 1), lambda i: (i, 0)),
            pl.BlockSpec((v.shape[0], _NUM_THETAS), lambda i: (0, 0)),
            pl.BlockSpec((1, st), lambda i: (0, 0)),
        ],
        out_specs=pl.BlockSpec((_NUM_SEGMENTS, st), lambda i: (0, 0)),
        out_shape=jax.ShapeDtypeStruct((_NUM_SEGMENTS, st), jnp.float32),
    )(x, batch2d, v_scaled, lin_row)
    return out.reshape(_NUM_SEGMENTS, _BUMP_STEPS, _NUM_THETAS)
